# Initial kernel scaffold; baseline (speedup 1.0000x reference)
import jax
import jax.numpy as jnp
from jax.experimental import pallas as pl


def kernel(features, edge_index, W, a_src, a_dst):
    raise NotImplementedError("write your pallas kernel here")


# speedup vs baseline: 11.4609x; 11.4609x over previous
import jax
import jax.numpy as jnp
from jax import lax
from jax.experimental import pallas as pl
from jax.experimental.pallas import tpu as pltpu
from jax.experimental.pallas import tpu_sc as plsc

N = 10000
D = 128
E = 320000
ALPHA = 0.2

NCORES = 2
NSUB = 16
NW = NCORES * NSUB
EB = E // NW
KB = 80
NCH = EB // KB
ACH = 2000
NACH = EB // ACH
DEN_R = 640


def _tc_stage(xin, W, a_s, a_d, combine, do_elu):
    def body(x_ref, w_ref, as_ref, ad_ref, h_ref, s_ref, t_ref, c_ref):
        if combine:
            x = x_ref[0] + x_ref[1]
        else:
            x = x_ref[...]
        if do_elu:
            x = jnp.where(x > 0.0, x, jnp.exp(x) - 1.0)
        h = jnp.dot(x, w_ref[...], preferred_element_type=jnp.float32)
        h_ref[...] = h
        s = jnp.sum(h * as_ref[...][None, :], axis=1)
        t = jnp.sum(h * ad_ref[...][None, :], axis=1)
        s_ref[...] = s
        t_ref[...] = t
        c = jnp.maximum(jnp.max(s) + jnp.max(t), 0.0)
        c_ref[...] = jnp.full((16,), c, jnp.float32)

    return pl.pallas_call(
        body,
        out_shape=[
            jax.ShapeDtypeStruct((N, D), jnp.float32),
            jax.ShapeDtypeStruct((N,), jnp.float32),
            jax.ShapeDtypeStruct((N,), jnp.float32),
            jax.ShapeDtypeStruct((16,), jnp.float32),
        ],
    )(xin, W, a_s, a_d)


def _tc_tail(parts):
    def body(x_ref, o_ref):
        x = x_ref[0] + x_ref[1]
        o_ref[...] = jnp.where(x > 0.0, x, jnp.exp(x) - 1.0)

    return pl.pallas_call(
        body, out_shape=jax.ShapeDtypeStruct((N, D), jnp.float32)
    )(parts)


def _sc_a_body(s_hbm, t_hbm, c_hbm, src_hbm, dst_hbm, ex_hbm, denp_hbm,
               s_v, t_v, den_v, c_v, ridx_v, srcb_v, dstb_v, exb_v,
               isem, den_sh):
    cid = lax.axis_index("c")
    sid = lax.axis_index("s")
    w = sid * NCORES + cid
    ebase = w * EB

    pltpu.sync_copy(s_hbm, s_v)
    pltpu.sync_copy(t_hbm, t_v)
    pltpu.sync_copy(c_hbm, c_v)
    cvec = c_v[...]

    zeros16 = jnp.zeros((16,), jnp.float32)
    iota16 = lax.iota(jnp.int32, 16)

    def _zden(j, _):
        den_v[j, :] = zeros16
        return 0
    lax.fori_loop(0, DEN_R, _zden, 0)
    for j in range(DEN_R // 16):
        ridx_v[j // 8, pl.ds((j % 8) * 16, 16)] = iota16 + j * 16

    def _issue_idx(i, b):
        off = ebase + i * ACH
        pltpu.async_copy(src_hbm.at[pl.ds(off, ACH)], srcb_v.at[b],
                         isem.at[b])
        pltpu.async_copy(dst_hbm.at[pl.ds(off, ACH)], dstb_v.at[b],
                         isem.at[b])

    def _wait_idx(i, b):
        off = ebase + i * ACH
        pltpu.make_async_copy(src_hbm.at[pl.ds(off, ACH)], srcb_v.at[b],
                              isem.at[b]).wait()
        pltpu.make_async_copy(dst_hbm.at[pl.ds(off, ACH)], dstb_v.at[b],
                              isem.at[b]).wait()

    _issue_idx(0, 0)
    for i in range(NACH):
        b = i % 2
        if i + 1 < NACH:
            _issue_idx(i + 1, (i + 1) % 2)
        _wait_idx(i, b)

        def _aloop(j, _):
            off = j * 16
            sidx = srcb_v[b, pl.ds(off, 16)]
            didx = dstb_v[b, pl.ds(off, 16)]
            sv = plsc.load_gather(s_v, [sidx])
            tv = plsc.load_gather(t_v, [didx])
            logit = sv + tv
            logit = jnp.where(logit >= 0.0, logit, ALPHA * logit)
            ex = jnp.exp(logit - cvec)
            exb_v[b, pl.ds(off, 16)] = ex
            r = lax.shift_right_logical(didx, 4)
            co = lax.bitwise_and(didx, 15)
            plsc.addupdate_scatter(den_v, [r, co], ex)
            return 0
        lax.fori_loop(0, ACH // 16, _aloop, 0)
        pltpu.sync_copy(exb_v.at[b], ex_hbm.at[pl.ds(ebase + i * ACH, ACH)])

    @pl.when(sid == 0)
    def _():
        pltpu.sync_copy(den_v, den_sh)
    plsc.subcore_barrier()

    @pl.when(sid != 0)
    def _():
        for i in range(DEN_R // 128):
            pltpu.sync_copy(den_v.at[pl.ds(i * 128, 128)],
                            den_sh.at[ridx_v.at[i]], add=True)
    plsc.subcore_barrier()

    rb = sid * (DEN_R // NSUB)
    pltpu.sync_copy(den_sh.at[pl.ds(rb, DEN_R // NSUB)],
                    denp_hbm.at[cid].at[pl.ds(rb, DEN_R // NSUB)])


def _sc_a(s, t, cvec, src, dst):
    mesh = plsc.VectorSubcoreMesh(core_axis_name="c", subcore_axis_name="s")
    kern = pl.kernel(
        _sc_a_body,
        out_type=[
            jax.ShapeDtypeStruct((E,), jnp.float32),
            jax.ShapeDtypeStruct((NCORES, DEN_R, 16), jnp.float32),
        ],
        mesh=mesh,
        compiler_params=pltpu.CompilerParams(
            needs_layout_passes=False, use_tc_tiling_on_sc=False),
        scratch_types=[
            pltpu.VMEM((N,), jnp.float32),
            pltpu.VMEM((N,), jnp.float32),
            pltpu.VMEM((DEN_R, 16), jnp.float32),
            pltpu.VMEM((16,), jnp.float32),
            pltpu.VMEM((DEN_R // 128, 128), jnp.int32),
            pltpu.VMEM((2, ACH), jnp.int32),
            pltpu.VMEM((2, ACH), jnp.int32),
            pltpu.VMEM((2, ACH), jnp.float32),
            pltpu.SemaphoreType.DMA((2,)),
            pltpu.VMEM_SHARED((DEN_R, 16), jnp.float32),
        ],
    )
    return kern(s, t, cvec, src, dst)


def _sc_b_body(h_hbm, ex_hbm, denp_hbm, src_hbm, dst_hbm, out_hbm,
               den_v, stg_v, rows_v, sidx_v, didx_v, exc_v,
               isem, gsem, ssem, out_sh):
    cid = lax.axis_index("c")
    sid = lax.axis_index("s")
    w = sid * NCORES + cid
    ebase = w * EB

    pltpu.sync_copy(denp_hbm.at[0], den_v)
    SR = 160
    for q in range(DEN_R // SR):
        pltpu.sync_copy(denp_hbm.at[1].at[pl.ds(q * SR, SR)], stg_v)

        def _cmb(j, qq=q):
            def f(j, _):
                den_v[qq * SR + j, :] = 1.0 / (
                    den_v[qq * SR + j, :] + stg_v[j, :] + 1e-16)
                return 0
            return f
        lax.fori_loop(0, SR, _cmb(q), 0)

    zeros16 = jnp.zeros((16,), jnp.float32)

    def _zrows(r, _):
        for l in range(D // 16):
            rows_v[0, r, pl.ds(l * 16, 16)] = zeros16
        return 0
    lax.fori_loop(0, KB, _zrows, 0)
    zbase = sid * 640

    @pl.when(sid != NSUB - 1)
    def _():
        for i in range(640 // KB):
            pltpu.sync_copy(rows_v.at[0], out_sh.at[pl.ds(zbase + i * KB, KB)])

    @pl.when(sid == NSUB - 1)
    def _():
        for i in range(400 // KB):
            pltpu.sync_copy(rows_v.at[0], out_sh.at[pl.ds(zbase + i * KB, KB)])

    plsc.subcore_barrier()

    def _issue_idx(i, b):
        off = ebase + i * KB
        pltpu.async_copy(src_hbm.at[pl.ds(off, KB)], sidx_v.at[b], isem.at[b])
        pltpu.async_copy(dst_hbm.at[pl.ds(off, KB)], didx_v.at[b], isem.at[b])
        pltpu.async_copy(ex_hbm.at[pl.ds(off, KB)], exc_v.at[b], isem.at[b])

    def _wait_idx(i, b):
        off = ebase + i * KB
        pltpu.make_async_copy(src_hbm.at[pl.ds(off, KB)], sidx_v.at[b],
                              isem.at[b]).wait()
        pltpu.make_async_copy(dst_hbm.at[pl.ds(off, KB)], didx_v.at[b],
                              isem.at[b]).wait()
        pltpu.make_async_copy(ex_hbm.at[pl.ds(off, KB)], exc_v.at[b],
                              isem.at[b]).wait()

    def _issue_gather(b):
        pltpu.async_copy(h_hbm.at[sidx_v.at[b]], rows_v.at[b], gsem.at[b])

    def _wait_gather(b):
        pltpu.make_async_copy(h_hbm.at[sidx_v.at[b]], rows_v.at[b],
                              gsem.at[b]).wait()

    def _issue_scatter(b):
        pltpu.async_copy(rows_v.at[b], out_sh.at[didx_v.at[b]], ssem.at[b],
                         add=True)

    def _wait_scatter(b):
        pltpu.make_async_copy(rows_v.at[b], out_sh.at[didx_v.at[b]],
                              ssem.at[b]).wait()

    _issue_idx(0, 0)
    _issue_idx(1, 1)
    _wait_idx(0, 0)
    _issue_gather(0)

    def _bloop(i, _):
        b = lax.rem(i, 3)
        b1 = lax.rem(i + 1, 3)
        b2 = lax.rem(i + 2, 3)

        @pl.when(i >= 1)
        def _():
            _wait_scatter(b2)

        @pl.when(i + 2 < NCH)
        def _():
            _issue_idx(i + 2, b2)

        _wait_gather(b)

        def _scale(g, _):
            didx = didx_v[b, pl.ds(g * 16, 16)]
            r = lax.shift_right_logical(didx, 4)
            co = lax.bitwise_and(didx, 15)
            rden = plsc.load_gather(den_v, [r, co])
            ex = exc_v[b, pl.ds(g * 16, 16)]
            av = ex * rden
            for rr in range(16):
                a = av[rr]
                row = g * 16 + rr
                for l in range(D // 16):
                    sl = pl.ds(l * 16, 16)
                    rows_v[b, row, sl] = rows_v[b, row, sl] * a
            return 0
        lax.fori_loop(0, KB // 16, _scale, 0)

        _issue_scatter(b)

        @pl.when(i + 1 < NCH)
        def _():
            _wait_idx(i + 1, b1)
            _issue_gather(b1)
        return 0
    lax.fori_loop(0, NCH, _bloop, 0)

    _wait_scatter((NCH - 1) % 3)

    plsc.subcore_barrier()
    wbase = sid * 640

    @pl.when(sid != NSUB - 1)
    def _():
        pltpu.sync_copy(out_sh.at[pl.ds(wbase, 640)],
                        out_hbm.at[cid].at[pl.ds(wbase, 640)])

    @pl.when(sid == NSUB - 1)
    def _():
        pltpu.sync_copy(out_sh.at[pl.ds(wbase, 400)],
                        out_hbm.at[cid].at[pl.ds(wbase, 400)])


def _sc_b(h, ex, denp, src, dst):
    mesh = plsc.VectorSubcoreMesh(core_axis_name="c", subcore_axis_name="s")
    kern = pl.kernel(
        _sc_b_body,
        out_type=jax.ShapeDtypeStruct((NCORES, N, D), jnp.float32),
        mesh=mesh,
        compiler_params=pltpu.CompilerParams(
            needs_layout_passes=False, use_tc_tiling_on_sc=False),
        scratch_types=[
            pltpu.VMEM((DEN_R, 16), jnp.float32),
            pltpu.VMEM((160, 16), jnp.float32),
            pltpu.VMEM((3, KB, D), jnp.float32),
            pltpu.VMEM((3, KB), jnp.int32),
            pltpu.VMEM((3, KB), jnp.int32),
            pltpu.VMEM((3, KB), jnp.float32),
            pltpu.SemaphoreType.DMA((3,)),
            pltpu.SemaphoreType.DMA((3,)),
            pltpu.SemaphoreType.DMA((3,)),
            pltpu.VMEM_SHARED((N, D), jnp.float32),
        ],
    )
    return kern(h, ex, denp, src, dst)


def _sc_conv(h, s, t, cvec, src, dst):
    ex, denp = _sc_a(s, t, cvec, src, dst)
    return _sc_b(h, ex, denp, src, dst)


def kernel(features, edge_index, W, a_src, a_dst):
    src = edge_index[0].astype(jnp.int32)
    dst = edge_index[1].astype(jnp.int32)

    h, s, t, c = _tc_stage(features, W[0], a_src[0], a_dst[0],
                           combine=False, do_elu=False)
    parts = _sc_conv(h, s, t, c, src, dst)
    h, s, t, c = _tc_stage(parts, W[1], a_src[1], a_dst[1],
                           combine=True, do_elu=False)
    parts = _sc_conv(h, s, t, c, src, dst)
    h, s, t, c = _tc_stage(parts, W[2], a_src[2], a_dst[2],
                           combine=True, do_elu=True)
    parts = _sc_conv(h, s, t, c, src, dst)
    h, s, t, c = _tc_stage(parts, W[3], a_src[3], a_dst[3],
                           combine=True, do_elu=False)
    parts = _sc_conv(h, s, t, c, src, dst)
    return _tc_tail(parts)


# speedup vs baseline: 14.7357x; 1.2857x over previous
import jax
import jax.numpy as jnp
from jax import lax
from jax.experimental import pallas as pl
from jax.experimental.pallas import tpu as pltpu
from jax.experimental.pallas import tpu_sc as plsc

N = 10000
D = 128
E = 320000
ALPHA = 0.2

NCORES = 2
NSUB = 16
NW = NCORES * NSUB
EB = E // NW
KB = 80
NCH = EB // KB
ACH = 2000
NACH = EB // ACH
DEN_R = 640


def _tc_stage(xin, W, a_s, a_d, combine, do_elu):
    def body(x_ref, w_ref, as_ref, ad_ref, h_ref, s_ref, t_ref, c_ref):
        if combine:
            x = x_ref[0] + x_ref[1]
        else:
            x = x_ref[...]
        if do_elu:
            x = jnp.where(x > 0.0, x, jnp.exp(x) - 1.0)
        h = jnp.dot(x, w_ref[...], preferred_element_type=jnp.float32)
        h_ref[...] = h
        s = jnp.sum(h * as_ref[...][None, :], axis=1)
        t = jnp.sum(h * ad_ref[...][None, :], axis=1)
        s_ref[...] = s
        t_ref[...] = t
        c = jnp.maximum(jnp.max(s) + jnp.max(t), 0.0)
        c_ref[...] = jnp.full((16,), c, jnp.float32)

    return pl.pallas_call(
        body,
        out_shape=[
            jax.ShapeDtypeStruct((N, D), jnp.float32),
            jax.ShapeDtypeStruct((N,), jnp.float32),
            jax.ShapeDtypeStruct((N,), jnp.float32),
            jax.ShapeDtypeStruct((16,), jnp.float32),
        ],
    )(xin, W, a_s, a_d)


def _tc_tail(parts):
    def body(x_ref, o_ref):
        x = x_ref[0] + x_ref[1]
        o_ref[...] = jnp.where(x > 0.0, x, jnp.exp(x) - 1.0)

    return pl.pallas_call(
        body, out_shape=jax.ShapeDtypeStruct((N, D), jnp.float32)
    )(parts)


def _sc_a_body(s_hbm, t_hbm, c_hbm, src_hbm, dst_hbm, ex_hbm, denp_hbm,
               s_v, t_v, den_v, c_v, ridx_v, srcb_v, dstb_v, exb_v,
               isem, den_sh):
    cid = lax.axis_index("c")
    sid = lax.axis_index("s")
    w = sid * NCORES + cid
    ebase = w * EB

    pltpu.sync_copy(s_hbm, s_v)
    pltpu.sync_copy(t_hbm, t_v)
    pltpu.sync_copy(c_hbm, c_v)
    cvec = c_v[...]

    zeros16 = jnp.zeros((16,), jnp.float32)
    iota16 = lax.iota(jnp.int32, 16)

    def _zden(j, _):
        den_v[j, :] = zeros16
        return 0
    lax.fori_loop(0, DEN_R, _zden, 0)
    for j in range(DEN_R // 16):
        ridx_v[j // 8, pl.ds((j % 8) * 16, 16)] = iota16 + j * 16

    def _issue_idx(i, b):
        off = ebase + i * ACH
        pltpu.async_copy(src_hbm.at[pl.ds(off, ACH)], srcb_v.at[b],
                         isem.at[b])
        pltpu.async_copy(dst_hbm.at[pl.ds(off, ACH)], dstb_v.at[b],
                         isem.at[b])

    def _wait_idx(i, b):
        off = ebase + i * ACH
        pltpu.make_async_copy(src_hbm.at[pl.ds(off, ACH)], srcb_v.at[b],
                              isem.at[b]).wait()
        pltpu.make_async_copy(dst_hbm.at[pl.ds(off, ACH)], dstb_v.at[b],
                              isem.at[b]).wait()

    _issue_idx(0, 0)
    for i in range(NACH):
        b = i % 2
        if i + 1 < NACH:
            _issue_idx(i + 1, (i + 1) % 2)
        _wait_idx(i, b)

        def _aloop(j, _):
            off = j * 16
            sidx = srcb_v[b, pl.ds(off, 16)]
            didx = dstb_v[b, pl.ds(off, 16)]
            sv = plsc.load_gather(s_v, [sidx])
            tv = plsc.load_gather(t_v, [didx])
            logit = sv + tv
            logit = jnp.where(logit >= 0.0, logit, ALPHA * logit)
            ex = jnp.exp(logit - cvec)
            exb_v[b, pl.ds(off, 16)] = ex
            r = lax.shift_right_logical(didx, 4)
            co = lax.bitwise_and(didx, 15)
            plsc.addupdate_scatter(den_v, [r, co], ex)
            return 0
        lax.fori_loop(0, ACH // 16, _aloop, 0)
        pltpu.sync_copy(exb_v.at[b], ex_hbm.at[pl.ds(ebase + i * ACH, ACH)])

    @pl.when(sid == 0)
    def _():
        pltpu.sync_copy(den_v, den_sh)
    plsc.subcore_barrier()

    @pl.when(sid != 0)
    def _():
        for i in range(DEN_R // 128):
            pltpu.sync_copy(den_v.at[pl.ds(i * 128, 128)],
                            den_sh.at[ridx_v.at[i]], add=True)
    plsc.subcore_barrier()

    rb = sid * (DEN_R // NSUB)
    pltpu.sync_copy(den_sh.at[pl.ds(rb, DEN_R // NSUB)],
                    denp_hbm.at[cid].at[pl.ds(rb, DEN_R // NSUB)])


def _sc_a(s, t, cvec, src, dst):
    mesh = plsc.VectorSubcoreMesh(core_axis_name="c", subcore_axis_name="s")
    kern = pl.kernel(
        _sc_a_body,
        out_type=[
            jax.ShapeDtypeStruct((E,), jnp.float32),
            jax.ShapeDtypeStruct((NCORES, DEN_R, 16), jnp.float32),
        ],
        mesh=mesh,
        compiler_params=pltpu.CompilerParams(
            needs_layout_passes=False, use_tc_tiling_on_sc=False),
        scratch_types=[
            pltpu.VMEM((N,), jnp.float32),
            pltpu.VMEM((N,), jnp.float32),
            pltpu.VMEM((DEN_R, 16), jnp.float32),
            pltpu.VMEM((16,), jnp.float32),
            pltpu.VMEM((DEN_R // 128, 128), jnp.int32),
            pltpu.VMEM((2, ACH), jnp.int32),
            pltpu.VMEM((2, ACH), jnp.int32),
            pltpu.VMEM((2, ACH), jnp.float32),
            pltpu.SemaphoreType.DMA((2,)),
            pltpu.VMEM_SHARED((DEN_R, 16), jnp.float32),
        ],
    )
    return kern(s, t, cvec, src, dst)


def _sc_b_body(h_hbm, ex_hbm, denp_hbm, src_hbm, dst_hbm, out_hbm,
               den_v, stg_v, rows_v, sidx_v, didx_v, exc_v,
               isem, gsem, ssem, out_sh):
    cid = lax.axis_index("c")
    sid = lax.axis_index("s")
    w = sid * NCORES + cid
    ebase = w * EB

    pltpu.sync_copy(denp_hbm.at[0], den_v)
    SR = 160
    for q in range(DEN_R // SR):
        pltpu.sync_copy(denp_hbm.at[1].at[pl.ds(q * SR, SR)], stg_v)

        def _cmb(j, qq=q):
            def f(j, _):
                den_v[qq * SR + j, :] = 1.0 / (
                    den_v[qq * SR + j, :] + stg_v[j, :] + 1e-16)
                return 0
            return f
        lax.fori_loop(0, SR, _cmb(q), 0)

    zeros16 = jnp.zeros((16,), jnp.float32)

    def _zrows(r, _):
        for l in range(D // 16):
            rows_v[0, r, pl.ds(l * 16, 16)] = zeros16
        return 0
    lax.fori_loop(0, KB, _zrows, 0)
    zbase = sid * 640

    @pl.when(sid != NSUB - 1)
    def _():
        for i in range(640 // KB):
            pltpu.sync_copy(rows_v.at[0], out_sh.at[pl.ds(zbase + i * KB, KB)])

    @pl.when(sid == NSUB - 1)
    def _():
        for i in range(400 // KB):
            pltpu.sync_copy(rows_v.at[0], out_sh.at[pl.ds(zbase + i * KB, KB)])

    plsc.subcore_barrier()

    def _issue_idx(i, b):
        off = ebase + i * KB
        pltpu.async_copy(src_hbm.at[pl.ds(off, KB)], sidx_v.at[b], isem.at[b])
        pltpu.async_copy(dst_hbm.at[pl.ds(off, KB)], didx_v.at[b], isem.at[b])
        pltpu.async_copy(ex_hbm.at[pl.ds(off, KB)], exc_v.at[b], isem.at[b])

    def _wait_idx(i, b):
        off = ebase + i * KB
        pltpu.make_async_copy(src_hbm.at[pl.ds(off, KB)], sidx_v.at[b],
                              isem.at[b]).wait()
        pltpu.make_async_copy(dst_hbm.at[pl.ds(off, KB)], didx_v.at[b],
                              isem.at[b]).wait()
        pltpu.make_async_copy(ex_hbm.at[pl.ds(off, KB)], exc_v.at[b],
                              isem.at[b]).wait()

    def _issue_gather(ib, rb):
        pltpu.async_copy(h_hbm.at[sidx_v.at[ib]], rows_v.at[rb], gsem.at[ib])

    def _wait_gather(ib, rb):
        pltpu.make_async_copy(h_hbm.at[sidx_v.at[ib]], rows_v.at[rb],
                              gsem.at[ib]).wait()

    def _issue_scatter(ib, rb):
        pltpu.async_copy(rows_v.at[rb], out_sh.at[didx_v.at[ib]], ssem.at[ib],
                         add=True)

    def _wait_scatter(ib, rb):
        pltpu.make_async_copy(rows_v.at[rb], out_sh.at[didx_v.at[ib]],
                              ssem.at[ib]).wait()

    _issue_idx(0, 0)
    _issue_idx(1, 1)
    _wait_idx(0, 0)
    _issue_gather(0, 0)

    def _bloop(i, _):
        rb = lax.rem(i, 3)
        rb1 = lax.rem(i + 1, 3)
        ib = lax.rem(i, 4)
        ib1 = lax.rem(i + 1, 4)
        ib2 = lax.rem(i + 2, 4)

        @pl.when(i + 2 < NCH)
        def _():
            _issue_idx(i + 2, ib2)

        @pl.when(i + 1 < NCH)
        def _():
            _wait_idx(i + 1, ib1)
            _issue_gather(ib1, rb1)

        _wait_gather(ib, rb)

        def _scale(g, _):
            didx = didx_v[ib, pl.ds(g * 16, 16)]
            r = lax.shift_right_logical(didx, 4)
            co = lax.bitwise_and(didx, 15)
            rden = plsc.load_gather(den_v, [r, co])
            ex = exc_v[ib, pl.ds(g * 16, 16)]
            av = ex * rden
            for rr in range(16):
                a = av[rr]
                row = g * 16 + rr
                for l in range(D // 16):
                    sl = pl.ds(l * 16, 16)
                    rows_v[rb, row, sl] = rows_v[rb, row, sl] * a
            return 0
        lax.fori_loop(0, KB // 16, _scale, 0)

        _issue_scatter(ib, rb)

        @pl.when(i >= 1)
        def _():
            _wait_scatter(lax.rem(i - 1, 4), lax.rem(i - 1, 3))
        return 0
    lax.fori_loop(0, NCH, _bloop, 0)

    _wait_scatter((NCH - 1) % 4, (NCH - 1) % 3)

    plsc.subcore_barrier()
    wbase = sid * 640

    @pl.when(sid != NSUB - 1)
    def _():
        pltpu.sync_copy(out_sh.at[pl.ds(wbase, 640)],
                        out_hbm.at[cid].at[pl.ds(wbase, 640)])

    @pl.when(sid == NSUB - 1)
    def _():
        pltpu.sync_copy(out_sh.at[pl.ds(wbase, 400)],
                        out_hbm.at[cid].at[pl.ds(wbase, 400)])


def _sc_b(h, ex, denp, src, dst):
    mesh = plsc.VectorSubcoreMesh(core_axis_name="c", subcore_axis_name="s")
    kern = pl.kernel(
        _sc_b_body,
        out_type=jax.ShapeDtypeStruct((NCORES, N, D), jnp.float32),
        mesh=mesh,
        compiler_params=pltpu.CompilerParams(
            needs_layout_passes=False, use_tc_tiling_on_sc=False),
        scratch_types=[
            pltpu.VMEM((DEN_R, 16), jnp.float32),
            pltpu.VMEM((160, 16), jnp.float32),
            pltpu.VMEM((3, KB, D), jnp.float32),
            pltpu.VMEM((4, KB), jnp.int32),
            pltpu.VMEM((4, KB), jnp.int32),
            pltpu.VMEM((4, KB), jnp.float32),
            pltpu.SemaphoreType.DMA((4,)),
            pltpu.SemaphoreType.DMA((4,)),
            pltpu.SemaphoreType.DMA((4,)),
            pltpu.VMEM_SHARED((N, D), jnp.float32),
        ],
    )
    return kern(h, ex, denp, src, dst)


def _sc_conv(h, s, t, cvec, src, dst):
    ex, denp = _sc_a(s, t, cvec, src, dst)
    return _sc_b(h, ex, denp, src, dst)


def kernel(features, edge_index, W, a_src, a_dst):
    src = edge_index[0].astype(jnp.int32)
    dst = edge_index[1].astype(jnp.int32)

    h, s, t, c = _tc_stage(features, W[0], a_src[0], a_dst[0],
                           combine=False, do_elu=False)
    parts = _sc_conv(h, s, t, c, src, dst)
    h, s, t, c = _tc_stage(parts, W[1], a_src[1], a_dst[1],
                           combine=True, do_elu=False)
    parts = _sc_conv(h, s, t, c, src, dst)
    h, s, t, c = _tc_stage(parts, W[2], a_src[2], a_dst[2],
                           combine=True, do_elu=True)
    parts = _sc_conv(h, s, t, c, src, dst)
    h, s, t, c = _tc_stage(parts, W[3], a_src[3], a_dst[3],
                           combine=True, do_elu=False)
    parts = _sc_conv(h, s, t, c, src, dst)
    return _tc_tail(parts)


# speedup vs baseline: 26.0982x; 1.7711x over previous
import jax
import jax.numpy as jnp
from jax import lax
from jax.experimental import pallas as pl
from jax.experimental.pallas import tpu as pltpu
from jax.experimental.pallas import tpu_sc as plsc

N = 10000
D = 128
E = 320000
ALPHA = 0.2

NCORES = 2
NSUB = 16
NW = NCORES * NSUB
EB = E // NW
KB = 80
NCH = EB // KB
ACH = 2000
NACH = EB // ACH
DEN_R = 640


def _tc_stage(xin, W, a_s, a_d, combine, do_elu):
    def body(x_ref, w_ref, as_ref, ad_ref, h_ref, s_ref, t_ref, c_ref):
        if combine:
            x = x_ref[0] + x_ref[1]
        else:
            x = x_ref[...]
        if do_elu:
            x = jnp.where(x > 0.0, x, jnp.exp(x) - 1.0)
        h = jnp.dot(x, w_ref[...], preferred_element_type=jnp.float32)
        h_ref[...] = h
        s = jnp.sum(h * as_ref[...][None, :], axis=1)
        t = jnp.sum(h * ad_ref[...][None, :], axis=1)
        s_ref[...] = s
        t_ref[...] = t
        c = jnp.maximum(jnp.max(s) + jnp.max(t), 0.0)
        c_ref[...] = jnp.full((16,), c, jnp.float32)

    return pl.pallas_call(
        body,
        out_shape=[
            jax.ShapeDtypeStruct((N, D), jnp.float32),
            jax.ShapeDtypeStruct((N,), jnp.float32),
            jax.ShapeDtypeStruct((N,), jnp.float32),
            jax.ShapeDtypeStruct((16,), jnp.float32),
        ],
    )(xin, W, a_s, a_d)


def _tc_tail(parts):
    def body(x_ref, o_ref):
        x = x_ref[0] + x_ref[1]
        o_ref[...] = jnp.where(x > 0.0, x, jnp.exp(x) - 1.0)

    return pl.pallas_call(
        body, out_shape=jax.ShapeDtypeStruct((N, D), jnp.float32)
    )(parts)


def _sc_a_body(s_hbm, t_hbm, c_hbm, src_hbm, dst_hbm, ex_hbm, denp_hbm,
               s_v, t_v, den_v, c_v, ridx_v, srcb_v, dstb_v, exb_v,
               isem, den_sh):
    cid = lax.axis_index("c")
    sid = lax.axis_index("s")
    w = sid * NCORES + cid
    ebase = w * EB

    pltpu.sync_copy(s_hbm, s_v)
    pltpu.sync_copy(t_hbm, t_v)
    pltpu.sync_copy(c_hbm, c_v)
    cvec = c_v[...]

    zeros16 = jnp.zeros((16,), jnp.float32)
    iota16 = lax.iota(jnp.int32, 16)

    def _zden(j, _):
        den_v[j, :] = zeros16
        return 0
    lax.fori_loop(0, DEN_R, _zden, 0)
    for j in range(DEN_R // 16):
        ridx_v[j // 8, pl.ds((j % 8) * 16, 16)] = iota16 + j * 16

    def _issue_idx(i, b):
        off = ebase + i * ACH
        pltpu.async_copy(src_hbm.at[pl.ds(off, ACH)], srcb_v.at[b],
                         isem.at[b])
        pltpu.async_copy(dst_hbm.at[pl.ds(off, ACH)], dstb_v.at[b],
                         isem.at[b])

    def _wait_idx(i, b):
        off = ebase + i * ACH
        pltpu.make_async_copy(src_hbm.at[pl.ds(off, ACH)], srcb_v.at[b],
                              isem.at[b]).wait()
        pltpu.make_async_copy(dst_hbm.at[pl.ds(off, ACH)], dstb_v.at[b],
                              isem.at[b]).wait()

    _issue_idx(0, 0)
    for i in range(NACH):
        b = i % 2
        if i + 1 < NACH:
            _issue_idx(i + 1, (i + 1) % 2)
        _wait_idx(i, b)

        def _aloop(j, _):
            off = j * 16
            sidx = srcb_v[b, pl.ds(off, 16)]
            didx = dstb_v[b, pl.ds(off, 16)]
            sv = plsc.load_gather(s_v, [sidx])
            tv = plsc.load_gather(t_v, [didx])
            logit = sv + tv
            logit = jnp.where(logit >= 0.0, logit, ALPHA * logit)
            ex = jnp.exp(logit - cvec)
            exb_v[b, pl.ds(off, 16)] = ex
            r = lax.shift_right_logical(didx, 4)
            co = lax.bitwise_and(didx, 15)
            plsc.addupdate_scatter(den_v, [r, co], ex)
            return 0
        lax.fori_loop(0, ACH // 16, _aloop, 0)
        pltpu.sync_copy(exb_v.at[b], ex_hbm.at[pl.ds(ebase + i * ACH, ACH)])

    @pl.when(sid == 0)
    def _():
        pltpu.sync_copy(den_v, den_sh)
    plsc.subcore_barrier()

    @pl.when(sid != 0)
    def _():
        for i in range(DEN_R // 128):
            pltpu.sync_copy(den_v.at[pl.ds(i * 128, 128)],
                            den_sh.at[ridx_v.at[i]], add=True)
    plsc.subcore_barrier()

    rb = sid * (DEN_R // NSUB)
    pltpu.sync_copy(den_sh.at[pl.ds(rb, DEN_R // NSUB)],
                    denp_hbm.at[cid].at[pl.ds(rb, DEN_R // NSUB)])


def _sc_a(s, t, cvec, src, dst):
    mesh = plsc.VectorSubcoreMesh(core_axis_name="c", subcore_axis_name="s")
    kern = pl.kernel(
        _sc_a_body,
        out_type=[
            jax.ShapeDtypeStruct((E,), jnp.float32),
            jax.ShapeDtypeStruct((NCORES, DEN_R, 16), jnp.float32),
        ],
        mesh=mesh,
        compiler_params=pltpu.CompilerParams(
            needs_layout_passes=False, use_tc_tiling_on_sc=False),
        scratch_types=[
            pltpu.VMEM((N,), jnp.float32),
            pltpu.VMEM((N,), jnp.float32),
            pltpu.VMEM((DEN_R, 16), jnp.float32),
            pltpu.VMEM((16,), jnp.float32),
            pltpu.VMEM((DEN_R // 128, 128), jnp.int32),
            pltpu.VMEM((2, ACH), jnp.int32),
            pltpu.VMEM((2, ACH), jnp.int32),
            pltpu.VMEM((2, ACH), jnp.float32),
            pltpu.SemaphoreType.DMA((2,)),
            pltpu.VMEM_SHARED((DEN_R, 16), jnp.float32),
        ],
    )
    return kern(s, t, cvec, src, dst)


def _sc_b_body(h_hbm, ex_hbm, denp_hbm, src_hbm, dst_hbm, out_hbm,
               den_v, stg_v, rows_v, sidx_v, didx_v, exc_v,
               isem, gsem, ssem, out_sh):
    cid = lax.axis_index("c")
    sid = lax.axis_index("s")
    w = sid * NCORES + cid
    ebase = w * EB

    pltpu.sync_copy(denp_hbm.at[0], den_v)
    SR = 160
    for q in range(DEN_R // SR):
        pltpu.sync_copy(denp_hbm.at[1].at[pl.ds(q * SR, SR)], stg_v)

        def _cmb(j, qq=q):
            def f(j, _):
                den_v[qq * SR + j, :] = 1.0 / (
                    den_v[qq * SR + j, :] + stg_v[j, :] + 1e-16)
                return 0
            return f
        lax.fori_loop(0, SR, _cmb(q), 0)

    zeros16 = jnp.zeros((16,), jnp.float32)

    def _zrows(r, _):
        for l in range(D // 16):
            rows_v[0, r, pl.ds(l * 16, 16)] = zeros16
        return 0
    lax.fori_loop(0, KB, _zrows, 0)
    zbase = sid * 640

    @pl.when(sid != NSUB - 1)
    def _():
        for i in range(640 // KB):
            pltpu.sync_copy(rows_v.at[0], out_sh.at[pl.ds(zbase + i * KB, KB)])

    @pl.when(sid == NSUB - 1)
    def _():
        for i in range(400 // KB):
            pltpu.sync_copy(rows_v.at[0], out_sh.at[pl.ds(zbase + i * KB, KB)])

    plsc.subcore_barrier()

    def _issue_idx(i, b):
        off = ebase + i * KB
        pltpu.async_copy(src_hbm.at[pl.ds(off, KB)], sidx_v.at[b], isem.at[b])
        pltpu.async_copy(dst_hbm.at[pl.ds(off, KB)], didx_v.at[b], isem.at[b])
        pltpu.async_copy(ex_hbm.at[pl.ds(off, KB)], exc_v.at[b], isem.at[b])

    def _wait_idx(i, b):
        off = ebase + i * KB
        pltpu.make_async_copy(src_hbm.at[pl.ds(off, KB)], sidx_v.at[b],
                              isem.at[b]).wait()
        pltpu.make_async_copy(dst_hbm.at[pl.ds(off, KB)], didx_v.at[b],
                              isem.at[b]).wait()
        pltpu.make_async_copy(ex_hbm.at[pl.ds(off, KB)], exc_v.at[b],
                              isem.at[b]).wait()

    def _issue_gather(ib, rb):
        pltpu.async_copy(h_hbm.at[sidx_v.at[ib]], rows_v.at[rb], gsem.at[ib])

    def _wait_gather(ib, rb):
        pltpu.make_async_copy(h_hbm.at[sidx_v.at[ib]], rows_v.at[rb],
                              gsem.at[ib]).wait()

    def _issue_scatter(ib, rb):
        pltpu.async_copy(rows_v.at[rb], out_sh.at[didx_v.at[ib]], ssem.at[ib],
                         add=True)

    def _wait_scatter(ib, rb):
        pltpu.make_async_copy(rows_v.at[rb], out_sh.at[didx_v.at[ib]],
                              ssem.at[ib]).wait()

    _issue_idx(0, 0)
    _issue_idx(1, 1)
    _wait_idx(0, 0)
    _issue_gather(0, 0)

    def _bloop(i, _):
        rb = lax.rem(i, 3)
        rb1 = lax.rem(i + 1, 3)
        ib = lax.rem(i, 4)
        ib1 = lax.rem(i + 1, 4)
        ib2 = lax.rem(i + 2, 4)

        @pl.when(i + 2 < NCH)
        def _():
            _issue_idx(i + 2, ib2)

        @pl.when(i + 1 < NCH)
        def _():
            _wait_idx(i + 1, ib1)
            _issue_gather(ib1, rb1)

        _wait_gather(ib, rb)

        rowb = rows_v.at[rb]
        didxb = didx_v.at[ib]
        excb = exc_v.at[ib]

        @plsc.parallel_loop(0, KB // 16, unroll=2)
        def _scale(g):
            didx = didxb[pl.ds(g * 16, 16)]
            r = lax.shift_right_logical(didx, 4)
            co = lax.bitwise_and(didx, 15)
            rden = plsc.load_gather(den_v, [r, co])
            ex = excb[pl.ds(g * 16, 16)]
            av = ex * rden
            for rr in range(16):
                a = av[rr]
                row = g * 16 + rr
                for l in range(D // 16):
                    sl = pl.ds(l * 16, 16)
                    rowb[row, sl] = rowb[row, sl] * a

        _issue_scatter(ib, rb)

        @pl.when(i >= 1)
        def _():
            _wait_scatter(lax.rem(i - 1, 4), lax.rem(i - 1, 3))
        return 0
    lax.fori_loop(0, NCH, _bloop, 0)

    _wait_scatter((NCH - 1) % 4, (NCH - 1) % 3)

    plsc.subcore_barrier()
    wbase = sid * 640

    @pl.when(sid != NSUB - 1)
    def _():
        pltpu.sync_copy(out_sh.at[pl.ds(wbase, 640)],
                        out_hbm.at[cid].at[pl.ds(wbase, 640)])

    @pl.when(sid == NSUB - 1)
    def _():
        pltpu.sync_copy(out_sh.at[pl.ds(wbase, 400)],
                        out_hbm.at[cid].at[pl.ds(wbase, 400)])


def _sc_b(h, ex, denp, src, dst):
    mesh = plsc.VectorSubcoreMesh(core_axis_name="c", subcore_axis_name="s")
    kern = pl.kernel(
        _sc_b_body,
        out_type=jax.ShapeDtypeStruct((NCORES, N, D), jnp.float32),
        mesh=mesh,
        compiler_params=pltpu.CompilerParams(
            needs_layout_passes=False, use_tc_tiling_on_sc=False),
        scratch_types=[
            pltpu.VMEM((DEN_R, 16), jnp.float32),
            pltpu.VMEM((160, 16), jnp.float32),
            pltpu.VMEM((3, KB, D), jnp.float32),
            pltpu.VMEM((4, KB), jnp.int32),
            pltpu.VMEM((4, KB), jnp.int32),
            pltpu.VMEM((4, KB), jnp.float32),
            pltpu.SemaphoreType.DMA((4,)),
            pltpu.SemaphoreType.DMA((4,)),
            pltpu.SemaphoreType.DMA((4,)),
            pltpu.VMEM_SHARED((N, D), jnp.float32),
        ],
    )
    return kern(h, ex, denp, src, dst)


def _sc_conv(h, s, t, cvec, src, dst):
    ex, denp = _sc_a(s, t, cvec, src, dst)
    return _sc_b(h, ex, denp, src, dst)


def kernel(features, edge_index, W, a_src, a_dst):
    src = edge_index[0].astype(jnp.int32)
    dst = edge_index[1].astype(jnp.int32)

    h, s, t, c = _tc_stage(features, W[0], a_src[0], a_dst[0],
                           combine=False, do_elu=False)
    parts = _sc_conv(h, s, t, c, src, dst)
    h, s, t, c = _tc_stage(parts, W[1], a_src[1], a_dst[1],
                           combine=True, do_elu=False)
    parts = _sc_conv(h, s, t, c, src, dst)
    h, s, t, c = _tc_stage(parts, W[2], a_src[2], a_dst[2],
                           combine=True, do_elu=True)
    parts = _sc_conv(h, s, t, c, src, dst)
    h, s, t, c = _tc_stage(parts, W[3], a_src[3], a_dst[3],
                           combine=True, do_elu=False)
    parts = _sc_conv(h, s, t, c, src, dst)
    return _tc_tail(parts)
